# baseline (speedup 1.0000x reference)
import jax
import jax.numpy as jnp
from jax.experimental import pallas as pl
from jax.experimental.pallas import tpu as pltpu

_ROWS = 64
_COLS = 8192
_CHUNK = 1024
_N = _COLS // _CHUNK
_INF = float("inf")


def _tc_body(xl_any, xu_any, outl_ref, outu_ref,
             bxl0, bxl1, bxu0, bxu1, cka, ckb, cxl, cxu,
             sl0, sl1, su0, su1):
    bxl = (bxl0, bxl1)
    bxu = (bxu0, bxu1)
    sl = (sl0, sl1)
    su = (su0, su1)

    def mkcp(i):
        cs = pl.ds(i * _CHUNK, _CHUNK)
        return (
            pltpu.make_async_copy(xl_any.at[:, cs], bxl[i % 2], sl[i % 2]),
            pltpu.make_async_copy(xu_any.at[:, cs], bxu[i % 2], su[i % 2]),
        )

    cps = [mkcp(i) for i in range(_N)]
    cps[0][0].start()
    cps[0][1].start()
    for i in range(_N):
        cps[i][0].wait()
        cps[i][1].wait()
        if i + 1 < _N:
            cps[i + 1][0].start()
            cps[i + 1][1].start()
        xl = bxl[i % 2][...]
        xu = bxu[i % 2][...]
        ka = jnp.float32(0.7) * xl + jnp.float32(0.3) * xu
        kb = jnp.float32(0.3) * xl + jnp.float32(0.7) * xu
        if i == 0:
            cka[...] = ka
            ckb[...] = kb
            cxl[...] = xl
            cxu[...] = xu
        else:
            bka = cka[...]
            bkb = ckb[...]
            better = (ka < bka) | ((ka == bka) & (kb < bkb))
            cka[...] = jnp.where(better, ka, bka)
            ckb[...] = jnp.where(better, kb, bkb)
            cxl[...] = jnp.where(better, xl, cxl[...])
            cxu[...] = jnp.where(better, xu, cxu[...])

    fka = cka[...]
    fkb = ckb[...]
    minka = jnp.min(fka, axis=1, keepdims=True)
    kbm = jnp.where(fka == minka, fkb, _INF)
    minkb = jnp.min(kbm, axis=1, keepdims=True)
    sel = kbm == minkb
    outl_ref[...] = jnp.min(jnp.where(sel, cxl[...], _INF), axis=1,
                            keepdims=True)
    outu_ref[...] = jnp.min(jnp.where(sel, cxu[...], _INF), axis=1,
                            keepdims=True)


@jax.jit
def kernel(xl, xu):
    return pl.pallas_call(
        _tc_body,
        in_specs=[
            pl.BlockSpec(memory_space=pl.ANY),
            pl.BlockSpec(memory_space=pl.ANY),
        ],
        out_shape=(
            jax.ShapeDtypeStruct((_ROWS, 1), jnp.float32),
            jax.ShapeDtypeStruct((_ROWS, 1), jnp.float32),
        ),
        scratch_shapes=[
            pltpu.VMEM((_ROWS, _CHUNK), jnp.float32),
            pltpu.VMEM((_ROWS, _CHUNK), jnp.float32),
            pltpu.VMEM((_ROWS, _CHUNK), jnp.float32),
            pltpu.VMEM((_ROWS, _CHUNK), jnp.float32),
            pltpu.VMEM((_ROWS, _CHUNK), jnp.float32),
            pltpu.VMEM((_ROWS, _CHUNK), jnp.float32),
            pltpu.VMEM((_ROWS, _CHUNK), jnp.float32),
            pltpu.VMEM((_ROWS, _CHUNK), jnp.float32),
            pltpu.SemaphoreType.DMA,
            pltpu.SemaphoreType.DMA,
            pltpu.SemaphoreType.DMA,
            pltpu.SemaphoreType.DMA,
        ],
    )(xl, xu)


# speedup vs baseline: 1.2754x; 1.2754x over previous
import jax
import jax.numpy as jnp
from jax.experimental import pallas as pl
from jax.experimental.pallas import tpu as pltpu

_ROWS = 64
_COLS = 8192
_RBLK = 8
_GRID = _ROWS // _RBLK
_INF = float("inf")


def _tc_body(xl_ref, xu_ref, outl_ref, outu_ref):
    xl = xl_ref[...]
    xu = xu_ref[...]
    ka = jnp.float32(0.7) * xl + jnp.float32(0.3) * xu
    minka = jnp.min(ka, axis=1, keepdims=True)
    kb = jnp.float32(0.3) * xl + jnp.float32(0.7) * xu
    kbm = jnp.where(ka == minka, kb, _INF)
    minkb = jnp.min(kbm, axis=1, keepdims=True)
    sel = kbm == minkb
    outl_ref[...] = jnp.min(jnp.where(sel, xl, _INF), axis=1, keepdims=True)
    outu_ref[...] = jnp.min(jnp.where(sel, xu, _INF), axis=1, keepdims=True)


@jax.jit
def kernel(xl, xu):
    return pl.pallas_call(
        _tc_body,
        grid=(_GRID,),
        in_specs=[
            pl.BlockSpec((_RBLK, _COLS), lambda i: (i, 0)),
            pl.BlockSpec((_RBLK, _COLS), lambda i: (i, 0)),
        ],
        out_specs=(
            pl.BlockSpec((_RBLK, 1), lambda i: (i, 0)),
            pl.BlockSpec((_RBLK, 1), lambda i: (i, 0)),
        ),
        out_shape=(
            jax.ShapeDtypeStruct((_ROWS, 1), jnp.float32),
            jax.ShapeDtypeStruct((_ROWS, 1), jnp.float32),
        ),
    )(xl, xu)


# speedup vs baseline: 1.8600x; 1.4584x over previous
import jax
import jax.numpy as jnp
from jax.experimental import pallas as pl
from jax.experimental.pallas import tpu as pltpu

_ROWS = 64
_COLS = 8192
_RBLK = 32
_GRID = _ROWS // _RBLK
_INF = float("inf")


def _tc_body(xl_ref, xu_ref, outl_ref, outu_ref):
    xl = xl_ref[...]
    xu = xu_ref[...]
    ka = jnp.float32(0.7) * xl + jnp.float32(0.3) * xu
    minka = jnp.min(ka, axis=1, keepdims=True)
    kb = jnp.float32(0.3) * xl + jnp.float32(0.7) * xu
    kbm = jnp.where(ka == minka, kb, _INF)
    minkb = jnp.min(kbm, axis=1, keepdims=True)
    sel = kbm == minkb
    outl_ref[...] = jnp.min(jnp.where(sel, xl, _INF), axis=1, keepdims=True)
    outu_ref[...] = jnp.min(jnp.where(sel, xu, _INF), axis=1, keepdims=True)


@jax.jit
def kernel(xl, xu):
    return pl.pallas_call(
        _tc_body,
        grid=(_GRID,),
        in_specs=[
            pl.BlockSpec((_RBLK, _COLS), lambda i: (i, 0)),
            pl.BlockSpec((_RBLK, _COLS), lambda i: (i, 0)),
        ],
        out_specs=(
            pl.BlockSpec((_RBLK, 1), lambda i: (i, 0)),
            pl.BlockSpec((_RBLK, 1), lambda i: (i, 0)),
        ),
        out_shape=(
            jax.ShapeDtypeStruct((_ROWS, 1), jnp.float32),
            jax.ShapeDtypeStruct((_ROWS, 1), jnp.float32),
        ),
    )(xl, xu)
